# baseline (speedup 1.0000x reference)
import functools

import jax
import jax.numpy as jnp
from jax import lax
from jax.experimental import pallas as pl
from jax.experimental.pallas import tpu as pltpu
from jax.experimental.pallas import tpu_sc as plsc

_EMBED_DIM = 64
_NW = 32
_CHUNK = 128
_NCHUNK = 8
_EPS = 1e-8


def _sc_gather_body(qidx_hbm, sidx_hbm, table_hbm, qrows_hbm, srows_hbm,
                    *scratch):
    idx_bufs = scratch[0:_NCHUNK]
    rows_v = scratch[_NCHUNK]
    sidx_v, sup_v, sem, sem_s = scratch[_NCHUNK + 1:]
    nw_rows = rows_v.shape[0]
    wid = lax.axis_index("s") * 2 + lax.axis_index("c")
    base = wid * nw_rows
    for j in range(_NCHUNK):
        pltpu.sync_copy(qidx_hbm.at[pl.ds(base + j * _CHUNK, _CHUNK)],
                        idx_bufs[j])
    copies = []
    for j in range(_NCHUNK):
        copies.append(pltpu.async_copy(
            table_hbm.at[idx_bufs[j]],
            rows_v.at[pl.ds(j * _CHUNK, _CHUNK)], sem))
    for c in copies:
        c.wait()
    pltpu.sync_copy(rows_v, qrows_hbm.at[pl.ds(base, nw_rows)])

    @pl.when(wid == 0)
    def _():
        pltpu.sync_copy(sidx_hbm, sidx_v)
        pltpu.async_copy(table_hbm.at[sidx_v], sup_v, sem_s).wait()
        pltpu.sync_copy(sup_v, srows_hbm)


def _tc_reduce_body(qe_ref, sup_ref, out_ref):
    sup = sup_ref[...]
    rid = lax.broadcasted_iota(jnp.int32, sup.shape, 0)
    m = jnp.sum(jnp.where(rid < 5, sup, 0.0), axis=0, keepdims=True) * 0.2
    n2 = jnp.maximum(jnp.sqrt(jnp.sum(m * m)), _EPS)
    x = qe_ref[...]
    num = jnp.sum(x * m[None], axis=2)
    sq = jnp.sum(x * x, axis=2)
    n1 = jnp.maximum(jnp.sqrt(sq), _EPS)
    out_ref[...] = num / (n1 * n2)


def kernel(query, support, symbol_emb):
    b = query.shape[0]
    n_rows = b * 2
    per_w = n_rows // _NW
    qidx = query.reshape(-1).astype(jnp.int32)
    sidx = jnp.concatenate(
        [support.reshape(-1).astype(jnp.int32),
         jnp.zeros((16 - 2 * support.shape[0],), jnp.int32)])

    gather = functools.partial(
        pl.kernel,
        mesh=plsc.VectorSubcoreMesh(core_axis_name="c", subcore_axis_name="s",
                                    num_cores=2),
        compiler_params=pltpu.CompilerParams(use_tc_tiling_on_sc=False),
        out_type=(jax.ShapeDtypeStruct((n_rows, _EMBED_DIM), jnp.float32),
                  jax.ShapeDtypeStruct((16, _EMBED_DIM), jnp.float32)),
        scratch_types=tuple([pltpu.VMEM((_CHUNK,), jnp.int32)] * _NCHUNK
                            + [pltpu.VMEM((per_w, _EMBED_DIM), jnp.float32),
                               pltpu.VMEM((16,), jnp.int32),
                               pltpu.VMEM((16, _EMBED_DIM), jnp.float32),
                               pltpu.SemaphoreType.DMA,
                               pltpu.SemaphoreType.DMA]),
    )(_sc_gather_body)
    qrows, srows = gather(qidx, sidx, symbol_emb)

    qe3 = qrows.reshape(b // 128, 128, 2 * _EMBED_DIM)
    sup8 = srows.reshape(8, 2 * _EMBED_DIM)
    bq = 16
    out2d = pl.pallas_call(
        _tc_reduce_body,
        grid=(b // 128 // bq,),
        in_specs=[pl.BlockSpec((bq, 128, 2 * _EMBED_DIM), lambda i: (i, 0, 0)),
                  pl.BlockSpec((8, 2 * _EMBED_DIM), lambda i: (0, 0))],
        out_specs=pl.BlockSpec((bq, 128), lambda i: (i, 0)),
        out_shape=jax.ShapeDtypeStruct((b // 128, 128), jnp.float32),
    )(qe3, sup8)
    return out2d.reshape(b)
